# baseline (speedup 1.0000x reference)
import functools

import jax
import jax.numpy as jnp
from jax.experimental import pallas as pl
from jax.experimental.pallas import tpu as pltpu

N = 512
LPAD = 131072


def _gcn_kernel(x_ref, wa_ref, w0_ref, b0_ref, w1_ref, b1_ref, w2_ref, b2_ref,
                o_ref, win_scr):
    LWA = N * (N - 1) // 2
    W = N + 128

    def body(ib, _):
        for t in range(32):
            i = ib * 32 + t
            s = i * (N - 2) - (i * (i - 1)) // 2 - 1
            base = pl.multiple_of(
                jnp.clip((s // 128) * 128, 0, LWA - W), 128)
            win_scr[pl.ds(i, 1), :] = wa_ref[0, pl.ds(base, W)].reshape(1, W)
        return 0

    jax.lax.fori_loop(0, N // 32, body, 0)

    ri = jax.lax.broadcasted_iota(jnp.int32, (N, 1), 0)
    sv = ri * (N - 2) - (ri * (ri - 1)) // 2 - 1
    qv = sv - jnp.clip((sv // 128) * 128, 0, LWA - W) + 1
    m = win_scr[:, :]
    for b in range(8):
        rolled = pltpu.roll(m, W - (1 << b), axis=1)
        m = jnp.where((qv & (1 << b)) != 0, rolled, m)
    vals = pltpu.roll(m, 1, axis=1)[:, :N]

    rows = jax.lax.broadcasted_iota(jnp.int32, (N, N), 0)
    cols = jax.lax.broadcasted_iota(jnp.int32, (N, N), 1)
    z = jnp.exp2(vals * jnp.float32(-1.4426950408889634))
    au = jnp.where(cols > rows, pl.reciprocal(1.0 + z, approx=True), 0.0)
    eye = jnp.where(rows == cols, 1.0, 0.0)
    a = au + au.T + eye

    deg = jnp.sum(a, axis=0, keepdims=True)
    dinv = jax.lax.rsqrt(deg)
    ahat = a * dinv * dinv.T

    def dot_w(a, w):
        return jnp.dot(a.astype(jnp.bfloat16), w.astype(jnp.bfloat16),
                       preferred_element_type=jnp.float32)

    dot_a = functools.partial(jnp.dot, preferred_element_type=jnp.float32)
    h = dot_w(x_ref[:, :], w0_ref[:, :])
    h = jax.nn.relu(dot_a(ahat, h) + b0_ref[0, :])
    h = dot_w(h, w1_ref[:, :])
    h = jax.nn.relu(dot_a(ahat, h) + b1_ref[0, :])
    h = dot_w(h, w2_ref[:, :])
    o_ref[:, :] = jax.nn.relu(dot_a(ahat, h) + b2_ref[0, :])


@jax.jit
def kernel(x, Wa, W0, b0, W1, b1, W2, b2):
    return pl.pallas_call(
        _gcn_kernel,
        out_shape=jax.ShapeDtypeStruct((N, W2.shape[1]), jnp.float32),
        scratch_shapes=[pltpu.VMEM((N, N + 128), jnp.float32)],
    )(x, Wa.reshape(1, N * (N - 1) // 2), W0, b0.reshape(1, -1),
      W1, b1.reshape(1, -1), W2, b2.reshape(1, -1))


# speedup vs baseline: 1.1005x; 1.1005x over previous
import functools

import jax
import jax.numpy as jnp
from jax.experimental import pallas as pl
from jax.experimental.pallas import tpu as pltpu

N = 512
LPAD = 131072


def _gcn_kernel(x_ref, wa_ref, w0_ref, b0_ref, w1_ref, b1_ref, w2_ref, b2_ref,
                o_ref, au_scr, t1_scr, t2_scr, t3_scr):
    LWA = N * (N - 1) // 2

    def run_tier(row0, scr):
        c0 = row0
        loadw = N - c0 + 128
        storew = N - c0

        def body(ib, _):
            for t in range(32):
                i = row0 + ib * 32 + t
                s = i * (N - 2) - (i * (i - 1)) // 2 - 1 + c0
                base = pl.multiple_of(
                    jnp.clip((s // 128) * 128, 0, LWA - loadw), 128)
                r = jnp.maximum(s - base, 0)
                win = wa_ref[0, pl.ds(base, loadw)].reshape(1, loadw)
                scr[pl.ds(i - row0, 1), :storew] = pltpu.roll(
                    win, loadw - r, axis=1)[:, :storew]
            return 0

        jax.lax.fori_loop(0, 128 // 32, body, 0)

    run_tier(0, au_scr)
    run_tier(128, t1_scr)
    run_tier(256, t2_scr)
    run_tier(384, t3_scr)
    au_scr[128:256, 128:N] = t1_scr[:, :]
    au_scr[256:384, 256:N] = t2_scr[:, :]
    au_scr[384:N, 384:N] = t3_scr[:, :]

    win0 = wa_ref[0, pl.ds(0, N + 128)].reshape(1, N + 128)
    au_scr[0:1, :] = pltpu.roll(win0, 1, axis=1)[:, :N]

    rows = jax.lax.broadcasted_iota(jnp.int32, (N, N), 0)
    cols = jax.lax.broadcasted_iota(jnp.int32, (N, N), 1)
    z = jnp.exp2(au_scr[:, :] * jnp.float32(-1.4426950408889634))
    au = jnp.where(cols > rows, pl.reciprocal(1.0 + z, approx=True), 0.0)
    eye = jnp.where(rows == cols, 1.0, 0.0)
    a = au + au.T + eye

    deg = jnp.sum(a, axis=0, keepdims=True)
    dinv = jax.lax.rsqrt(deg)
    ahat = a * dinv * dinv.T

    def dot_w(a, w):
        return jnp.dot(a.astype(jnp.bfloat16), w.astype(jnp.bfloat16),
                       preferred_element_type=jnp.float32)

    dot_a = functools.partial(jnp.dot, preferred_element_type=jnp.float32)
    h = dot_w(x_ref[:, :], w0_ref[:, :])
    h = jax.nn.relu(dot_a(ahat, h) + b0_ref[0, :])
    h = dot_w(h, w1_ref[:, :])
    h = jax.nn.relu(dot_a(ahat, h) + b1_ref[0, :])
    h = dot_w(h, w2_ref[:, :])
    o_ref[:, :] = jax.nn.relu(dot_a(ahat, h) + b2_ref[0, :])


@jax.jit
def kernel(x, Wa, W0, b0, W1, b1, W2, b2):
    return pl.pallas_call(
        _gcn_kernel,
        out_shape=jax.ShapeDtypeStruct((N, W2.shape[1]), jnp.float32),
        scratch_shapes=[pltpu.VMEM((N, N), jnp.float32),
                        pltpu.VMEM((128, 384), jnp.float32),
                        pltpu.VMEM((128, 256), jnp.float32),
                        pltpu.VMEM((128, 128), jnp.float32)],
    )(x, Wa.reshape(1, N * (N - 1) // 2), W0, b0.reshape(1, -1),
      W1, b1.reshape(1, -1), W2, b2.reshape(1, -1))
